# baseline (speedup 1.0000x reference)
import functools

import jax
import jax.numpy as jnp
from jax import lax
from jax.experimental import pallas as pl
from jax.experimental.pallas import tpu as pltpu
from jax.experimental.pallas import tpu_sc as plsc

B = 16384
F_CAT = 26
VOCAB = 100000
EMB = 32
NC = 2
NS = 16
NW = NC * NS
PAIRS = F_CAT * EMB
PAIRS_W = PAIRS // NW
BC = 1024
NCHUNK = B // BC
UNROLL = 16
LANES = 16

_mesh = plsc.VectorSubcoreMesh(core_axis_name="c", subcore_axis_name="s")


@functools.partial(
    pl.kernel,
    out_type=jax.ShapeDtypeStruct((PAIRS, B), jnp.float32),
    mesh=_mesh,
    scratch_types=[
        pltpu.VMEM((VOCAB,), jnp.float32),
        pltpu.VMEM((2, BC), jnp.int32),
        pltpu.VMEM((2, BC), jnp.float32),
        pltpu.VMEM_SHARED((2, B), jnp.int32),
        pltpu.SemaphoreType.DMA,
        pltpu.SemaphoreType.DMA,
        pltpu.SemaphoreType.DMA,
        pltpu.SemaphoreType.DMA,
    ],
    compiler_params=pltpu.CompilerParams(needs_layout_passes=False),
)
def _gather_kernel(tab_hbm, idx_hbm, out_hbm, row_v, idx_v, out_v, sidx, rsem, isem, osem, ssem):
    wid = lax.axis_index("s") * NC + lax.axis_index("c")
    sid = lax.axis_index("s")

    @pl.when(sid == 0)
    def _():
        pltpu.async_copy(idx_hbm.at[0], sidx.at[0], ssem)

    def pair_body(p, carry):
        g = wid + NW * p
        rcp = pltpu.async_copy(tab_hbm.at[g], row_v, rsem)
        pslot = jnp.bitwise_and(p, 1)
        @pl.when(sid == 0)
        def _():
            pltpu.make_async_copy(idx_hbm.at[0], sidx.at[0], ssem).wait()
        plsc.subcore_barrier()
        @pl.when(jnp.logical_and(sid == 0, p < PAIRS_W - 1))
        def _():
            pltpu.async_copy(idx_hbm.at[p + 1], sidx.at[1 - pslot], ssem)
        pltpu.async_copy(sidx.at[pslot, pl.ds(0, BC)], idx_v.at[0], isem)
        for c in range(NCHUNK):
            slot = c % 2
            pltpu.make_async_copy(
                sidx.at[0, pl.ds(0, BC)], idx_v.at[slot], isem
            ).wait()
            if c == 0:
                rcp.wait()
            if c >= 2:
                pltpu.make_async_copy(
                    out_hbm.at[0, pl.ds(0, BC)], out_v.at[slot], osem
                ).wait()
            else:
                @pl.when(p >= 1)
                def _():
                    pltpu.make_async_copy(
                        out_hbm.at[0, pl.ds(0, BC)], out_v.at[slot], osem
                    ).wait()
            if c < NCHUNK - 1:
                pltpu.async_copy(
                    sidx.at[pslot, pl.ds((c + 1) * BC, BC)], idx_v.at[1 - slot], isem
                )

            def gather_body(i, carry2):
                base = i * UNROLL * LANES
                for u in range(UNROLL):
                    sl = pl.ds(base + u * LANES, LANES)
                    out_v[slot, sl] = plsc.load_gather(row_v, [idx_v[slot, sl]])
                return carry2

            lax.fori_loop(0, BC // (UNROLL * LANES), gather_body, 0)
            pltpu.async_copy(out_v.at[slot], out_hbm.at[g, pl.ds(c * BC, BC)], osem)
        return carry

    lax.fori_loop(0, PAIRS_W, pair_body, 0)
    for slot in range(2):
        pltpu.make_async_copy(
            out_hbm.at[0, pl.ds(0, BC)], out_v.at[slot], osem
        ).wait()


def kernel(x_num_in, x_cat_in, tables):
    tab = tables.transpose(0, 2, 1).reshape(PAIRS, VOCAB)
    idx = x_cat_in.T
    out = _gather_kernel(tab, idx)
    x_cats = out.reshape(F_CAT, EMB, B).transpose(2, 0, 1)
    return (x_num_in, x_cats)


# speedup vs baseline: 1.0949x; 1.0949x over previous
import functools

import jax
import jax.numpy as jnp
from jax import lax
from jax.experimental import pallas as pl
from jax.experimental.pallas import tpu as pltpu
from jax.experimental.pallas import tpu_sc as plsc

B = 16384
F_CAT = 26
VOCAB = 100000
EMB = 32
NC = 2
NS = 16
NW = NC * NS
PAIRS = F_CAT * EMB
PAIRS_W = PAIRS // NW
BC = 2048
NCHUNK = B // BC
UNROLL = 16
LANES = 16

_mesh = plsc.VectorSubcoreMesh(core_axis_name="c", subcore_axis_name="s")


@functools.partial(
    pl.kernel,
    out_type=jax.ShapeDtypeStruct((PAIRS, B), jnp.float32),
    mesh=_mesh,
    scratch_types=[
        pltpu.VMEM((VOCAB,), jnp.float32),
        pltpu.VMEM((2, BC), jnp.int32),
        pltpu.VMEM((2, BC), jnp.float32),
        pltpu.VMEM_SHARED((2, B), jnp.int32),
        pltpu.SemaphoreType.DMA,
        pltpu.SemaphoreType.DMA,
        pltpu.SemaphoreType.DMA,
        pltpu.SemaphoreType.DMA,
    ],
    compiler_params=pltpu.CompilerParams(needs_layout_passes=False),
)
def _gather_kernel(tab_hbm, idx_hbm, out_hbm, row_v, idx_v, out_v, sidx, rsem, isem, osem, ssem):
    wid = lax.axis_index("s") * NC + lax.axis_index("c")
    sid = lax.axis_index("s")

    @pl.when(sid == 0)
    def _():
        pltpu.async_copy(idx_hbm.at[0], sidx.at[0], ssem)

    def pair_body(p, carry):
        g = wid + NW * p
        rcp = pltpu.async_copy(tab_hbm.at[g], row_v, rsem)
        pslot = jnp.bitwise_and(p, 1)
        @pl.when(sid == 0)
        def _():
            pltpu.make_async_copy(idx_hbm.at[0], sidx.at[0], ssem).wait()
        plsc.subcore_barrier()
        @pl.when(jnp.logical_and(sid == 0, p < PAIRS_W - 1))
        def _():
            pltpu.async_copy(idx_hbm.at[p + 1], sidx.at[1 - pslot], ssem)
        pltpu.async_copy(sidx.at[pslot, pl.ds(0, BC)], idx_v.at[0], isem)
        for c in range(NCHUNK):
            slot = c % 2
            pltpu.make_async_copy(
                sidx.at[0, pl.ds(0, BC)], idx_v.at[slot], isem
            ).wait()
            if c == 0:
                rcp.wait()
            if c >= 2:
                pltpu.make_async_copy(
                    out_hbm.at[0, pl.ds(0, BC)], out_v.at[slot], osem
                ).wait()
            else:
                @pl.when(p >= 1)
                def _():
                    pltpu.make_async_copy(
                        out_hbm.at[0, pl.ds(0, BC)], out_v.at[slot], osem
                    ).wait()
            if c < NCHUNK - 1:
                pltpu.async_copy(
                    sidx.at[pslot, pl.ds((c + 1) * BC, BC)], idx_v.at[1 - slot], isem
                )

            def gather_body(i, carry2):
                base = i * UNROLL * LANES
                for u in range(UNROLL):
                    sl = pl.ds(base + u * LANES, LANES)
                    out_v[slot, sl] = plsc.load_gather(row_v, [idx_v[slot, sl]])
                return carry2

            lax.fori_loop(0, BC // (UNROLL * LANES), gather_body, 0)
            pltpu.async_copy(out_v.at[slot], out_hbm.at[g, pl.ds(c * BC, BC)], osem)
        return carry

    lax.fori_loop(0, PAIRS_W, pair_body, 0)
    for slot in range(2):
        pltpu.make_async_copy(
            out_hbm.at[0, pl.ds(0, BC)], out_v.at[slot], osem
        ).wait()


def kernel(x_num_in, x_cat_in, tables):
    tab = tables.transpose(0, 2, 1).reshape(PAIRS, VOCAB)
    idx = x_cat_in.T
    out = _gather_kernel(tab, idx)
    x_cats = out.reshape(F_CAT, EMB, B).transpose(2, 0, 1)
    return (x_num_in, x_cats)
